# baseline (speedup 1.0000x reference)
import functools

import jax
import jax.numpy as jnp
from jax import lax
from jax.experimental import pallas as pl
from jax.experimental.pallas import tpu as pltpu
from jax.experimental.pallas import tpu_sc as plsc

_N = 10000
_NP = 10240
_E = 160000
_H = 160
_NB = 1000
_K = 80
_NC = 2
_NS = 16
_RPS = _NP // _NS
_CAP = 12800
_GRID = _N // _NB
_F32 = jnp.float32


def _mesh():
    return plsc.VectorSubcoreMesh(
        core_axis_name="c", subcore_axis_name="s",
        num_cores=_NC, num_subcores=_NS)


def _sc_embed(nf0s, nf1p, ntab0, nt1s):
    def body(nf0s, nf1p, ntab0, nt1s, out, nt1, rows, ridx, nf1m, sem):
        c = lax.axis_index("c")
        s = lax.axis_index("s")
        pltpu.sync_copy(nt1s.at[pl.ds(c * 8, 8)], nt1)
        base = s * _RPS

        def _chunk(i, carry):
            nb = base + i * _K
            pltpu.sync_copy(nf0s.at[pl.ds(c * _NP + nb, _K)], ridx)
            pltpu.sync_copy(nf1p.at[pl.ds(nb, _K)], nf1m)
            pltpu.async_copy(ntab0.at[ridx], rows, sem).wait()

            def _group(g, c2):
                uvec = nf1m[pl.ds(g * 16, 16)]
                for k in range(16):
                    u = uvec[k]
                    j = g * 16 + k
                    for t in range(_H // 16):
                        sl = pl.ds(t * 16, 16)
                        rows[j, sl] = rows[j, sl] + nt1[u, sl]
                return c2

            lax.fori_loop(0, _K // 16, _group, 0)
            pltpu.sync_copy(rows, out.at[pl.ds(c * _NP + nb, _K)])
            return carry

        lax.fori_loop(0, _RPS // _K, _chunk, 0)

    return pl.kernel(
        body,
        out_type=jax.ShapeDtypeStruct((2 * _NP, _H), _F32),
        mesh=_mesh(),
        compiler_params=pltpu.CompilerParams(use_tc_tiling_on_sc=False),
        scratch_types=[
            pltpu.VMEM((8, _H), _F32),
            pltpu.VMEM((_K, _H), _F32),
            pltpu.VMEM((_K,), jnp.int32),
            pltpu.VMEM((_K,), jnp.int32),
            pltpu.SemaphoreType.DMA,
        ],
    )(nf0s, nf1p, ntab0, nt1s)


def _sc_segsum(hs, srcs, dlb, eub, etabs, zrows):
    def body(hs, srcs, dlb, eub, etabs, zrows, out, acc, et, rows, ridx,
             dm, em, sem):
        c = lax.axis_index("c")
        s = lax.axis_index("s")
        pltpu.sync_copy(etabs.at[pl.ds(c * 32, 32)], et)
        pltpu.sync_copy(zrows, acc)
        ebase = s * _CAP

        def _chunk(i, carry):
            off = ebase + i * _K
            pltpu.sync_copy(srcs.at[pl.ds(c * (16 * _CAP) + off, _K)], ridx)
            pltpu.sync_copy(dlb.at[pl.ds(off, _K)], dm)
            pltpu.sync_copy(eub.at[pl.ds(off, _K)], em)
            pltpu.async_copy(hs.at[ridx], rows, sem).wait()

            def _group(g, c2):
                dvec = dm[pl.ds(g * 16, 16)]
                uvec = em[pl.ds(g * 16, 16)]
                for k in range(16):
                    d = dvec[k]
                    u = uvec[k]
                    j = g * 16 + k
                    for t in range(_H // 16):
                        sl = pl.ds(t * 16, 16)
                        acc[d, sl] = acc[d, sl] + (rows[j, sl] + et[u, sl])
                return c2

            lax.fori_loop(0, _K // 16, _group, 0)
            return carry

        lax.fori_loop(0, _CAP // _K, _chunk, 0)
        pltpu.sync_copy(acc.at[pl.ds(0, _RPS)],
                        out.at[pl.ds(c * _NP + s * _RPS, _RPS)])

    return pl.kernel(
        body,
        out_type=jax.ShapeDtypeStruct((2 * _NP, _H), _F32),
        mesh=_mesh(),
        compiler_params=pltpu.CompilerParams(use_tc_tiling_on_sc=False),
        scratch_types=[
            pltpu.VMEM((_RPS + 8, _H), _F32),
            pltpu.VMEM((32, _H), _F32),
            pltpu.VMEM((_K, _H), _F32),
            pltpu.VMEM((_K,), jnp.int32),
            pltpu.VMEM((_K,), jnp.int32),
            pltpu.VMEM((_K,), jnp.int32),
            pltpu.SemaphoreType.DMA,
        ],
    )(hs, srcs, dlb, eub, etabs, zrows)


def _cat300(ref):
    return jnp.concatenate([ref[0][:, :150], ref[1][:, :150]], axis=1)


def _mlp_body(hs, ss, w1, b1, w2, b2, z):
    agg = _cat300(hs) + _cat300(ss)
    z1 = jnp.dot(agg, w1[...], preferred_element_type=_F32) + b1[...]
    y = jnp.maximum(z1, 0.0)
    z[...] = jnp.dot(y, w2[...], preferred_element_type=_F32) + b2[...]


def _mlp(h3, s3, w1, b1, w2, b2):
    return pl.pallas_call(
        _mlp_body,
        grid=(_GRID,),
        in_specs=[
            pl.BlockSpec((2, _NB, _H), lambda i: (0, i, 0)),
            pl.BlockSpec((2, _NB, _H), lambda i: (0, i, 0)),
            pl.BlockSpec((300, 600), lambda i: (0, 0)),
            pl.BlockSpec((1, 600), lambda i: (0, 0)),
            pl.BlockSpec((600, 300), lambda i: (0, 0)),
            pl.BlockSpec((1, 300), lambda i: (0, 0)),
        ],
        out_specs=pl.BlockSpec((_NB, 300), lambda i: (i, 0)),
        out_shape=jax.ShapeDtypeStruct((_N, 300), _F32),
    )(h3, s3, w1, b1, w2, b2)


def _bn_body(z, mu, var, gam, bet, out, *, relu):
    zz = (z[...] - mu[...]) / jnp.sqrt(var[...] + 1e-5) * gam[...] + bet[...]
    if relu:
        zz = jnp.maximum(zz, 0.0)
    pad = jnp.zeros((_NB, 10), _F32)
    out[0] = jnp.concatenate([zz[:, :150], pad], axis=1)
    out[1] = jnp.concatenate([zz[:, 150:], pad], axis=1)


def _bn(z, mu, var, gam, bet, relu):
    return pl.pallas_call(
        functools.partial(_bn_body, relu=relu),
        grid=(_GRID,),
        in_specs=[
            pl.BlockSpec((_NB, 300), lambda i: (i, 0)),
            pl.BlockSpec((1, 300), lambda i: (0, 0)),
            pl.BlockSpec((1, 300), lambda i: (0, 0)),
            pl.BlockSpec((1, 300), lambda i: (0, 0)),
            pl.BlockSpec((1, 300), lambda i: (0, 0)),
        ],
        out_specs=pl.BlockSpec((2, _NB, _H), lambda i: (0, i, 0)),
        out_shape=jax.ShapeDtypeStruct((2, _NP, _H), _F32),
    )(z, mu, var, gam, bet)


def _readout_body(h3, n2g, wa, ba, wb, bb, wc, bc, out, gacc, cacc):
    i = pl.program_id(0)

    @pl.when(i == 0)
    def _():
        gacc[...] = jnp.zeros_like(gacc)
        cacc[...] = jnp.zeros_like(cacc)

    iot = lax.broadcasted_iota(jnp.int32, (_NB, 128), 1)
    oh = (iot == n2g[...]).astype(_F32)
    dnum = (((0,), (0,)), ((), ()))
    hcat = _cat300(h3)
    gacc[...] = gacc[...] + lax.dot_general(
        oh, hcat, dnum, preferred_element_type=_F32,
        precision=lax.Precision.HIGHEST)
    cacc[...] = cacc[...] + lax.dot_general(
        oh, jnp.ones((_NB, 8), _F32), dnum, preferred_element_type=_F32,
        precision=lax.Precision.HIGHEST)

    @pl.when(i == _GRID - 1)
    def _():
        cnt = jnp.maximum(cacc[:, 0:1], 1.0)
        g = gacc[...] / cnt
        a = jnp.maximum(jnp.dot(g, wa[...], preferred_element_type=_F32)
                        + ba[...], 0.0)
        a = jnp.maximum(jnp.dot(a, wb[...], preferred_element_type=_F32)
                        + bb[...], 0.0)
        o = jnp.dot(a, wc[...], preferred_element_type=_F32) + bc[...]
        out[...] = o[:64, 0:1]


def _readout(h3, n2g, wa, ba, wb, bb, wc, bc):
    return pl.pallas_call(
        _readout_body,
        grid=(_GRID,),
        in_specs=[
            pl.BlockSpec((2, _NB, _H), lambda i: (0, i, 0)),
            pl.BlockSpec((_NB, 1), lambda i: (i, 0)),
            pl.BlockSpec((300, 128), lambda i: (0, 0)),
            pl.BlockSpec((1, 128), lambda i: (0, 0)),
            pl.BlockSpec((128, 32), lambda i: (0, 0)),
            pl.BlockSpec((1, 32), lambda i: (0, 0)),
            pl.BlockSpec((32, 128), lambda i: (0, 0)),
            pl.BlockSpec((1, 128), lambda i: (0, 0)),
        ],
        out_specs=pl.BlockSpec((64, 1), lambda i: (0, 0)),
        out_shape=jax.ShapeDtypeStruct((64, 1), _F32),
        scratch_shapes=[
            pltpu.VMEM((128, 300), _F32),
            pltpu.VMEM((128, 8), _F32),
        ],
    )(h3, n2g, wa, ba, wb, bb, wc, bc)


def _halves(m, width=_H):
    r = m.shape[0]
    o = jnp.zeros((2, r, width), _F32)
    o = o.at[0, :, :150].set(m[:, :150])
    o = o.at[1, :, :150].set(m[:, 150:300])
    return o


def kernel(params, nfeats, efeats, edge_index, node2graph):
    nfeats = nfeats.astype(jnp.int32)
    efeats = efeats.astype(jnp.int32)
    edge_index = edge_index.astype(jnp.int32)
    node2graph = node2graph.astype(jnp.int32)

    ntab0 = _halves(jnp.concatenate(
        [params['node_emb0'], jnp.zeros((8, 300), _F32)], axis=0)
    ).reshape(256, _H)
    nt1s = _halves(jnp.concatenate(
        [params['node_emb1'], jnp.zeros((5, 300), _F32)], axis=0)
    ).reshape(16, _H)
    nf0p = jnp.zeros((_NP,), jnp.int32).at[:_N].set(nfeats[:, 0])
    nf0s = jnp.concatenate([nf0p, nf0p + 128])
    nf1p = jnp.zeros((_NP,), jnp.int32).at[:_N].set(nfeats[:, 1])

    src = edge_index[0]
    dst = edge_index[1]
    b = dst // _RPS
    order = jnp.argsort(b, stable=True)
    sso = src[order]
    dd = dst[order]
    uu = (efeats[:, 0] * 3 + efeats[:, 1])[order]
    bs = b[order]
    start = jnp.searchsorted(bs, jnp.arange(16, dtype=bs.dtype))
    pos = jnp.arange(_E) - start[bs]
    flat = bs * _CAP + pos
    srcb = jnp.zeros((16 * _CAP,), jnp.int32).at[flat].set(sso, mode='drop')
    dlb = jnp.full((16 * _CAP,), _RPS, jnp.int32).at[flat].set(
        dd - bs * _RPS, mode='drop')
    eub = jnp.full((16 * _CAP,), 31, jnp.int32).at[flat].set(uu, mode='drop')
    srcs = jnp.concatenate([srcb, srcb + _NP])
    zrows = jnp.zeros((_RPS + 8, _H), _F32)

    hs = _sc_embed(nf0s, nf1p, ntab0, nt1s)
    h3 = hs.reshape(2, _NP, _H)

    for l in range(5):
        p = params['layers'][l]
        comb = (p['edge_emb0'][:, None, :]
                + p['edge_emb1'][None, :, :]).reshape(18, 300)
        etabs = _halves(jnp.concatenate(
            [comb, jnp.zeros((14, 300), _F32)], axis=0)).reshape(64, _H)
        s3 = _sc_segsum(hs, srcs, dlb, eub, etabs, zrows).reshape(2, _NP, _H)
        z = _mlp(h3, s3, p['W1'], p['b1'].reshape(1, 600),
                 p['W2'], p['b2'].reshape(1, 300))
        mu = jnp.mean(z, axis=0, keepdims=True)
        var = jnp.var(z, axis=0, keepdims=True)
        h3 = _bn(z, mu, var, p['gamma'].reshape(1, 300),
                 p['beta'].reshape(1, 300), relu=(l < 4))
        hs = h3.reshape(2 * _NP, _H)

    wc = jnp.zeros((32, 128), _F32).at[:, 0].set(params['Wc'][:, 0])
    bc = jnp.zeros((1, 128), _F32).at[0, 0].set(params['bc'][0])
    return _readout(h3, node2graph.reshape(_N, 1),
                    params['Wa'], params['ba'].reshape(1, 128),
                    params['Wb'], params['bb'].reshape(1, 32), wc, bc)
